# baseline (speedup 1.0000x reference)
import functools

import jax
import jax.numpy as jnp
from jax.experimental import pallas as pl

_N = 10000
_E = 160000
_HEADS = 3


def _mm_body(x_ref, w_ref, b_ref, o_ref):
    o_ref[...] = (
        jnp.dot(x_ref[...], w_ref[...], preferred_element_type=jnp.float32)
        + b_ref[...]
    )


def _matmul_bias(x, w, b, block_rows=400):
    n, d = x.shape
    dout = w.shape[1]
    grid = (n // block_rows,)
    return pl.pallas_call(
        _mm_body,
        grid=grid,
        in_specs=[
            pl.BlockSpec((block_rows, d), lambda i: (i, 0)),
            pl.BlockSpec((d, dout), lambda i: (0, 0)),
            pl.BlockSpec((1, dout), lambda i: (0, 0)),
        ],
        out_specs=pl.BlockSpec((block_rows, dout), lambda i: (i, 0)),
        out_shape=jax.ShapeDtypeStruct((n, dout), jnp.float32),
    )(x, w, b.reshape(1, dout))


def _gatv2(x, edge_index, edge_attr, W, b, We, att, bias, heads, out_dim):
    src = edge_index[0]
    dst = edge_index[1]
    h = _matmul_bias(x, W, b).reshape(x.shape[0], heads, out_dim)
    e = (edge_attr @ We).reshape(-1, heads, out_dim)
    m = h[src] + h[dst] + e
    m = jnp.where(m >= 0, m, 0.2 * m)
    logits = jnp.sum(m * att[None, :, :], axis=-1)
    mx = jax.ops.segment_max(logits, dst, num_segments=_N)
    mx = jnp.where(jnp.isfinite(mx), mx, 0.0)
    ex = jnp.exp(logits - mx[dst])
    den = jax.ops.segment_sum(ex, dst, num_segments=_N)
    alpha = ex / (den[dst] + 1e-16)
    out = jax.ops.segment_sum(alpha[:, :, None] * h[src], dst, num_segments=_N)
    return out.reshape(_N, heads * out_dim) + bias


def kernel(x, edge_index_same_residue, edge_attr_same_residue,
           edge_index_same_protein, edge_attr_same_protein,
           W1, b1, We1, att1, bias1, W2, b2, We2, att2, bias2):
    x_orig = x
    h = _gatv2(x, edge_index_same_residue, edge_attr_same_residue,
               W1, b1, We1, att1, bias1, _HEADS, 64)
    h = jax.nn.relu(h)
    h = jnp.concatenate([h, x_orig], axis=1)
    h2 = _gatv2(h, edge_index_same_protein, edge_attr_same_protein,
                W2, b2, We2, att2, bias2, _HEADS, 160)
    h2 = jax.nn.relu(h2)
    return jnp.concatenate([h2, x_orig], axis=1)


# speedup vs baseline: 1.0693x; 1.0693x over previous
import functools

import jax
import jax.numpy as jnp
from jax import lax
from jax.experimental import pallas as pl
from jax.experimental.pallas import tpu as pltpu
from jax.experimental.pallas import tpu_sc as plsc

_N = 10000
_E = 160000
_HEADS = 3

_NC = 2
_NS = 16
_NW = _NC * _NS


def _mm_body(x_ref, w_ref, b_ref, o_ref):
    o_ref[...] = (
        jnp.dot(x_ref[...], w_ref[...], preferred_element_type=jnp.float32)
        + b_ref[...]
    )


def _matmul_bias(x, w, b, block_rows=400):
    n, d = x.shape
    dout = w.shape[1]
    grid = (n // block_rows,)
    return pl.pallas_call(
        _mm_body,
        grid=grid,
        in_specs=[
            pl.BlockSpec((block_rows, d), lambda i: (i, 0)),
            pl.BlockSpec((d, dout), lambda i: (0, 0)),
            pl.BlockSpec((1, dout), lambda i: (0, 0)),
        ],
        out_specs=pl.BlockSpec((block_rows, dout), lambda i: (i, 0)),
        out_shape=jax.ShapeDtypeStruct((n, dout), jnp.float32),
    )(x, w, b.reshape(1, dout))


def _make_edge_gather(C):
    K = 128 if C <= 256 else 112
    epw = _E // _NW
    nchunk = (epw + K - 1) // K
    mesh = plsc.VectorSubcoreMesh(core_axis_name="c", subcore_axis_name="s")

    @functools.partial(
        pl.kernel,
        mesh=mesh,
        out_type=[
            jax.ShapeDtypeStruct((_E, C), jnp.float32),
            jax.ShapeDtypeStruct((_E, C), jnp.float32),
        ],
        scratch_types=[
            pltpu.VMEM((K,), jnp.int32),
            pltpu.VMEM((K, C), jnp.float32),
            pltpu.SemaphoreType.DMA,
        ],
    )
    def gk(h_hbm, src_hbm, dst_hbm, hs_out, hd_out, idx, rows, sem):
        wid = lax.axis_index("s") * _NC + lax.axis_index("c")
        base = wid * epw
        last = base + epw - K

        def body(j, carry):
            off = jnp.minimum(base + j * K, last)
            pltpu.sync_copy(src_hbm.at[pl.ds(off, K)], idx)
            pltpu.async_copy(h_hbm.at[idx], rows, sem).wait()
            pltpu.sync_copy(rows, hs_out.at[pl.ds(off, K)])
            pltpu.sync_copy(dst_hbm.at[pl.ds(off, K)], idx)
            pltpu.async_copy(h_hbm.at[idx], rows, sem).wait()
            pltpu.sync_copy(rows, hd_out.at[pl.ds(off, K)])
            return carry

        lax.fori_loop(0, nchunk, body, 0)

    return gk


def _gatv2_v2(x, edge_index, edge_attr, W, b, We, att, bias, heads, out_dim):
    src = edge_index[0]
    dst = edge_index[1]
    C = heads * out_dim
    CP = ((C + 127) // 128) * 128
    Wp = jnp.pad(W, ((0, 0), (0, CP - C)))
    bp = jnp.pad(b, (0, CP - C))
    h = _matmul_bias(x, Wp, bp)
    hs, hd = _make_edge_gather(CP)(h, src, dst)
    hs = hs[:, :C]
    hd = hd[:, :C]
    e = (edge_attr @ We).reshape(-1, heads, out_dim)
    m = hs.reshape(-1, heads, out_dim) + hd.reshape(-1, heads, out_dim) + e
    m = jnp.where(m >= 0, m, 0.2 * m)
    logits = jnp.sum(m * att[None, :, :], axis=-1)
    mx = jax.ops.segment_max(logits, dst, num_segments=_N)
    mx = jnp.where(jnp.isfinite(mx), mx, 0.0)
    ex = jnp.exp(logits - mx[dst])
    den = jax.ops.segment_sum(ex, dst, num_segments=_N)
    alpha = ex / (den[dst] + 1e-16)
    out = jax.ops.segment_sum(
        alpha[:, :, None] * hs.reshape(-1, heads, out_dim), dst, num_segments=_N)
    return out.reshape(_N, heads * out_dim) + bias


def kernel(x, edge_index_same_residue, edge_attr_same_residue,
           edge_index_same_protein, edge_attr_same_protein,
           W1, b1, We1, att1, bias1, W2, b2, We2, att2, bias2):
    x_orig = x
    h = _gatv2_v2(x, edge_index_same_residue, edge_attr_same_residue,
                  W1, b1, We1, att1, bias1, _HEADS, 64)
    h = jax.nn.relu(h)
    h = jnp.concatenate([h, x_orig], axis=1)
    h2 = _gatv2_v2(h, edge_index_same_protein, edge_attr_same_protein,
                   W2, b2, We2, att2, bias2, _HEADS, 160)
    h2 = jax.nn.relu(h2)
    return jnp.concatenate([h2, x_orig], axis=1)


# speedup vs baseline: 1.0861x; 1.0157x over previous
import functools

import jax
import jax.numpy as jnp
from jax import lax
from jax.experimental import pallas as pl
from jax.experimental.pallas import tpu as pltpu
from jax.experimental.pallas import tpu_sc as plsc

_N = 10000
_E = 160000
_HEADS = 3

_NC = 2
_NS = 16
_NW = _NC * _NS


def _mm_body(x_ref, w_ref, b_ref, o_ref):
    o_ref[...] = (
        jnp.dot(x_ref[...], w_ref[...], preferred_element_type=jnp.float32)
        + b_ref[...]
    )


def _matmul_bias(x, w, b, block_rows=400):
    n, d = x.shape
    dout = w.shape[1]
    grid = (n // block_rows,)
    return pl.pallas_call(
        _mm_body,
        grid=grid,
        in_specs=[
            pl.BlockSpec((block_rows, d), lambda i: (i, 0)),
            pl.BlockSpec((d, dout), lambda i: (0, 0)),
            pl.BlockSpec((1, dout), lambda i: (0, 0)),
        ],
        out_specs=pl.BlockSpec((block_rows, dout), lambda i: (i, 0)),
        out_shape=jax.ShapeDtypeStruct((n, dout), jnp.float32),
    )(x, w, b.reshape(1, dout))


def _make_edge_gather(C):
    K = 128 if C <= 256 else 112
    epw = _E // _NW
    nchunk = (epw + K - 1) // K
    mesh = plsc.VectorSubcoreMesh(core_axis_name="c", subcore_axis_name="s")

    @functools.partial(
        pl.kernel,
        mesh=mesh,
        out_type=[
            jax.ShapeDtypeStruct((_E, C), jnp.float32),
            jax.ShapeDtypeStruct((_E, C), jnp.float32),
        ],
        scratch_types=[
            pltpu.VMEM((K,), jnp.int32),
            pltpu.VMEM((K, C), jnp.float32),
            pltpu.SemaphoreType.DMA,
        ],
    )
    def gk(h_hbm, src_hbm, dst_hbm, hs_out, hd_out, idx, rows, sem):
        wid = lax.axis_index("s") * _NC + lax.axis_index("c")
        base = wid * epw
        last = base + epw - K

        def body(j, carry):
            off = jnp.minimum(base + j * K, last)
            pltpu.sync_copy(src_hbm.at[pl.ds(off, K)], idx)
            pltpu.async_copy(h_hbm.at[idx], rows, sem).wait()
            pltpu.sync_copy(rows, hs_out.at[pl.ds(off, K)])
            pltpu.sync_copy(dst_hbm.at[pl.ds(off, K)], idx)
            pltpu.async_copy(h_hbm.at[idx], rows, sem).wait()
            pltpu.sync_copy(rows, hd_out.at[pl.ds(off, K)])
            return carry

        lax.fori_loop(0, nchunk, body, 0)

    return gk


def _make_vec_gather():
    K = 128
    C = 128
    epw = _E // _NW
    nchunk = (epw + K - 1) // K
    mesh = plsc.VectorSubcoreMesh(core_axis_name="c", subcore_axis_name="s")

    @functools.partial(
        pl.kernel,
        mesh=mesh,
        out_type=jax.ShapeDtypeStruct((_E, C), jnp.float32),
        scratch_types=[
            pltpu.VMEM((K,), jnp.int32),
            pltpu.VMEM((K, C), jnp.float32),
            pltpu.SemaphoreType.DMA,
        ],
    )
    def gk(t_hbm, dst_hbm, out_hbm, idx, rows, sem):
        wid = lax.axis_index("s") * _NC + lax.axis_index("c")
        base = wid * epw
        last = base + epw - K

        def body(j, carry):
            off = jnp.minimum(base + j * K, last)
            pltpu.sync_copy(dst_hbm.at[pl.ds(off, K)], idx)
            pltpu.async_copy(t_hbm.at[idx], rows, sem).wait()
            pltpu.sync_copy(rows, out_hbm.at[pl.ds(off, K)])
            return carry

        lax.fori_loop(0, nchunk, body, 0)

    return gk


def _gatv2_v2(x, edge_index, edge_attr, W, b, We, att, bias, heads, out_dim):
    src = edge_index[0]
    dst = edge_index[1]
    C = heads * out_dim
    CP = ((C + 127) // 128) * 128
    Wp = jnp.pad(W, ((0, 0), (0, CP - C)))
    bp = jnp.pad(b, (0, CP - C))
    h = _matmul_bias(x, Wp, bp)
    hs, hd = _make_edge_gather(CP)(h, src, dst)
    hs = hs[:, :C]
    hd = hd[:, :C]
    e = (edge_attr @ We).reshape(-1, heads, out_dim)
    m = hs.reshape(-1, heads, out_dim) + hd.reshape(-1, heads, out_dim) + e
    m = jnp.where(m >= 0, m, 0.2 * m)
    logits = jnp.sum(m * att[None, :, :], axis=-1)
    mx = jax.ops.segment_max(logits, dst, num_segments=_N)
    mx = jnp.where(jnp.isfinite(mx), mx, 0.0)
    mxg = _make_vec_gather()(jnp.pad(mx, ((0, 0), (0, 125))), dst)[:, :3]
    ex = jnp.exp(logits - mxg)
    den = jax.ops.segment_sum(ex, dst, num_segments=_N)
    deng = _make_vec_gather()(jnp.pad(den, ((0, 0), (0, 125))), dst)[:, :3]
    alpha = ex / (deng + 1e-16)
    out = jax.ops.segment_sum(
        alpha[:, :, None] * hs.reshape(-1, heads, out_dim), dst, num_segments=_N)
    return out.reshape(_N, heads * out_dim) + bias


def kernel(x, edge_index_same_residue, edge_attr_same_residue,
           edge_index_same_protein, edge_attr_same_protein,
           W1, b1, We1, att1, bias1, W2, b2, We2, att2, bias2):
    x_orig = x
    h = _gatv2_v2(x, edge_index_same_residue, edge_attr_same_residue,
                  W1, b1, We1, att1, bias1, _HEADS, 64)
    h = jax.nn.relu(h)
    h = jnp.concatenate([h, x_orig], axis=1)
    h2 = _gatv2_v2(h, edge_index_same_protein, edge_attr_same_protein,
                   W2, b2, We2, att2, bias2, _HEADS, 160)
    h2 = jax.nn.relu(h2)
    return jnp.concatenate([h2, x_orig], axis=1)
